# baseline (speedup 1.0000x reference)
import functools

import jax
import jax.numpy as jnp
from jax import lax
from jax.experimental import pallas as pl
from jax.experimental.pallas import tpu as pltpu
from jax.experimental.pallas import tpu_sc as plsc

EPS = 1e-12

_NUM_CORES = 2
_NUM_SUBCORES = 16
_NUM_WORKERS = _NUM_CORES * _NUM_SUBCORES
_CHUNK = 128


_NBUF = 5


def _sc_gather(table, idx2d):
    (n,) = idx2d.shape
    chunk = _CHUNK
    d = table.shape[1]
    dt = table.dtype
    rows_pw = n // _NUM_WORKERS
    chunks_pw = rows_pw // chunk
    n_rounds = chunks_pw // _NBUF
    mesh = plsc.VectorSubcoreMesh(core_axis_name="c", subcore_axis_name="s")

    @functools.partial(
        pl.kernel,
        out_type=jax.ShapeDtypeStruct((n, d), dt),
        mesh=mesh,
        scratch_types=[
            pltpu.VMEM((rows_pw,), jnp.int32),
            [pltpu.VMEM((chunk, d), dt) for _ in range(_NBUF)],
            [pltpu.SemaphoreType.DMA for _ in range(_NBUF)],
            [pltpu.SemaphoreType.DMA for _ in range(_NBUF)],
        ],
    )
    def gather_kernel(table_hbm, idx_hbm, out_hbm, idx_v, rows, gsem, wsem):
        wid = lax.axis_index("s") * _NUM_CORES + lax.axis_index("c")
        rbase = wid * rows_pw

        pltpu.sync_copy(idx_hbm.at[pl.ds(rbase, rows_pw)], idx_v)

        def g_start(c, b):
            pltpu.async_copy(table_hbm.at[idx_v.at[pl.ds(c * chunk, chunk)]],
                             rows[b], gsem[b])

        def g_wait(b):
            pltpu.make_async_copy(table_hbm.at[idx_v.at[pl.ds(0, chunk)]],
                                  rows[b], gsem[b]).wait()

        def w_start(c, b):
            pltpu.async_copy(
                rows[b], out_hbm.at[pl.ds(rbase + c * chunk, chunk)], wsem[b])

        def w_wait(b):
            pltpu.make_async_copy(rows[b], out_hbm.at[pl.ds(rbase, chunk)],
                                  wsem[b]).wait()

        for b in range(_NBUF - 1):
            g_start(b, b)
        w_start(_NBUF - 1, _NBUF - 1)

        def round_body(r, carry):
            c0 = r * _NBUF
            for b in range(_NBUF):
                c = c0 + b
                pb = (b + _NBUF - 1) % _NBUF
                w_wait(pb)
                g_start(c + _NBUF - 1, pb)
                g_wait(b)
                w_start(c, b)
            return carry

        lax.fori_loop(0, n_rounds - 1, round_body, 0)

        c0 = (n_rounds - 1) * _NBUF
        w_wait(_NBUF - 1)
        g_start(c0 + _NBUF - 1, _NBUF - 1)
        for b in range(_NBUF):
            g_wait(b)
            w_start(c0 + b, b)
        for b in range(_NBUF):
            w_wait(b)

    return gather_kernel(table, idx2d)


def _tc_dense(feats, gathered, w1, b1, w2, b2, ln_w, ln_b, batch, seq):
    f = feats.shape[-1]
    d = w1.shape[1]
    block_b = 256
    block = block_b * seq
    seq_pad = (seq + 7) // 8 * 8
    grid = (batch // block_b,)

    def body(f_ref, g_ref, w1_ref, b1_ref, w2_ref, b2_ref, lw_ref, lb_ref,
             o_ref):
        h = jnp.tanh(
            jnp.dot(f_ref[...], w1_ref[...],
                    preferred_element_type=jnp.float32) + b1_ref[...])
        nh = jnp.dot(h, w2_ref[...],
                     preferred_element_type=jnp.float32) + b2_ref[...]
        x = g_ref[...] + nh
        u = jnp.mean(x, axis=-1, keepdims=True)
        xc = x - u
        s = jnp.mean(xc * xc, axis=-1, keepdims=True)
        y = lw_ref[...] * (xc * lax.rsqrt(s + EPS)) + lb_ref[...]
        y3 = y.reshape(block_b, seq, d)
        pad = jnp.zeros((block_b, seq_pad - seq, d), jnp.float32)
        o_ref[...] = jnp.concatenate([y3, pad], axis=1)

    fixed = lambda i: (0, 0)
    return pl.pallas_call(
        body,
        grid=grid,
        in_specs=[
            pl.BlockSpec((block, f), lambda i: (i, 0)),
            pl.BlockSpec((block, d), lambda i: (i, 0)),
            pl.BlockSpec((f, d), fixed),
            pl.BlockSpec((1, d), fixed),
            pl.BlockSpec((d, d), fixed),
            pl.BlockSpec((1, d), fixed),
            pl.BlockSpec((1, d), fixed),
            pl.BlockSpec((1, d), fixed),
        ],
        out_specs=pl.BlockSpec((block_b, seq_pad, d), lambda i: (i, 0, 0)),
        out_shape=jax.ShapeDtypeStruct((batch, seq_pad, d), jnp.float32),
    )(feats, gathered, w1, b1, w2, b2, ln_w, ln_b)


def kernel(node_ids, node_features, emb_table, W1, b1, W2, b2, ln_weight,
           ln_bias):
    batch, seq = node_ids.shape
    n = batch * seq
    d = emb_table.shape[1]
    idx2d = node_ids.reshape(n).astype(jnp.int32)
    gathered = _sc_gather(emb_table, idx2d)
    feats = node_features.astype(jnp.bfloat16).reshape(n, -1)
    out = _tc_dense(feats, gathered, W1.astype(jnp.bfloat16),
                    b1.reshape(1, d), W2, b2.reshape(1, d),
                    ln_weight.reshape(1, d), ln_bias.reshape(1, d), batch,
                    seq)
    return lax.slice(out, (0, 0, 0), (batch, seq, d))


# speedup vs baseline: 1.0719x; 1.0719x over previous
import functools

import jax
import jax.numpy as jnp
from jax import lax
from jax.experimental import pallas as pl
from jax.experimental.pallas import tpu as pltpu
from jax.experimental.pallas import tpu_sc as plsc

EPS = 1e-12

_NUM_CORES = 2
_NUM_SUBCORES = 16
_NUM_WORKERS = _NUM_CORES * _NUM_SUBCORES
_CHUNK = 128


_NBUF = 5


def _sc_gather(table, idx2d):
    (n,) = idx2d.shape
    chunk = _CHUNK
    d = table.shape[1]
    dt = table.dtype
    rows_pw = n // _NUM_WORKERS
    chunks_pw = rows_pw // chunk
    n_rounds = chunks_pw // _NBUF
    mesh = plsc.VectorSubcoreMesh(core_axis_name="c", subcore_axis_name="s")

    @functools.partial(
        pl.kernel,
        out_type=jax.ShapeDtypeStruct((n, d), dt),
        mesh=mesh,
        scratch_types=[
            pltpu.VMEM((rows_pw,), jnp.int32),
            [pltpu.VMEM((chunk, d), dt) for _ in range(_NBUF)],
            [pltpu.SemaphoreType.DMA for _ in range(_NBUF)],
            [pltpu.SemaphoreType.DMA for _ in range(_NBUF)],
        ],
    )
    def gather_kernel(table_hbm, idx_hbm, out_hbm, idx_v, rows, gsem, wsem):
        wid = lax.axis_index("s") * _NUM_CORES + lax.axis_index("c")
        rbase = wid * rows_pw

        pltpu.sync_copy(idx_hbm.at[pl.ds(rbase, rows_pw)], idx_v)

        def g_start(c, b):
            pltpu.async_copy(table_hbm.at[idx_v.at[pl.ds(c * chunk, chunk)]],
                             rows[b], gsem[b])

        def g_wait(b):
            pltpu.make_async_copy(table_hbm.at[idx_v.at[pl.ds(0, chunk)]],
                                  rows[b], gsem[b]).wait()

        def w_start(c, b):
            pltpu.async_copy(
                rows[b], out_hbm.at[pl.ds(rbase + c * chunk, chunk)], wsem[b])

        def w_wait(b):
            pltpu.make_async_copy(rows[b], out_hbm.at[pl.ds(rbase, chunk)],
                                  wsem[b]).wait()

        for b in range(_NBUF - 1):
            g_start(b, b)
        w_start(_NBUF - 1, _NBUF - 1)

        def round_body(r, carry):
            c0 = r * _NBUF
            for b in range(_NBUF):
                c = c0 + b
                pb = (b + _NBUF - 1) % _NBUF
                w_wait(pb)
                g_start(c + _NBUF - 1, pb)
                g_wait(b)
                w_start(c, b)
            return carry

        lax.fori_loop(0, n_rounds - 1, round_body, 0)

        c0 = (n_rounds - 1) * _NBUF
        w_wait(_NBUF - 1)
        g_start(c0 + _NBUF - 1, _NBUF - 1)
        for b in range(_NBUF):
            g_wait(b)
            w_start(c0 + b, b)
        for b in range(_NBUF):
            w_wait(b)

    return gather_kernel(table, idx2d)


def _tc_dense(feats, gathered, w1, b1, w2, b2, ln_w, ln_b, batch, seq):
    f = feats.shape[-1]
    d = w1.shape[1]
    block_b = 256
    block = block_b * seq
    grid = (batch // block_b,)

    def body(f_ref, g_ref, w1_ref, b1_ref, w2_ref, b2_ref, lw_ref, lb_ref,
             o_ref):
        h = jnp.tanh(
            jnp.dot(f_ref[...].reshape(block, f), w1_ref[...],
                    preferred_element_type=jnp.float32) + b1_ref[...])
        nh = jnp.dot(h, w2_ref[...],
                     preferred_element_type=jnp.float32) + b2_ref[...]
        x = g_ref[...] + nh
        u = jnp.mean(x, axis=-1, keepdims=True)
        xc = x - u
        s = jnp.mean(xc * xc, axis=-1, keepdims=True)
        y = lw_ref[...] * (xc * lax.rsqrt(s + EPS)) + lb_ref[...]
        o_ref[...] = y.reshape(block_b, seq, d)

    fixed = lambda i: (0, 0)
    return pl.pallas_call(
        body,
        grid=grid,
        in_specs=[
            pl.BlockSpec((block_b, seq, f), lambda i: (i, 0, 0)),
            pl.BlockSpec((block, d), lambda i: (i, 0)),
            pl.BlockSpec((f, d), fixed),
            pl.BlockSpec((1, d), fixed),
            pl.BlockSpec((d, d), fixed),
            pl.BlockSpec((1, d), fixed),
            pl.BlockSpec((1, d), fixed),
            pl.BlockSpec((1, d), fixed),
        ],
        out_specs=pl.BlockSpec((block_b, seq, d), lambda i: (i, 0, 0)),
        out_shape=jax.ShapeDtypeStruct((batch, seq, d), jnp.float32),
    )(feats, gathered, w1, b1, w2, b2, ln_w, ln_b)


def kernel(node_ids, node_features, emb_table, W1, b1, W2, b2, ln_weight,
           ln_bias):
    batch, seq = node_ids.shape
    n = batch * seq
    d = emb_table.shape[1]
    idx2d = node_ids.reshape(n).astype(jnp.int32)
    gathered = _sc_gather(emb_table, idx2d)
    return _tc_dense(node_features.astype(jnp.bfloat16), gathered,
                     W1.astype(jnp.bfloat16), b1.reshape(1, d), W2,
                     b2.reshape(1, d), ln_weight.reshape(1, d),
                     ln_bias.reshape(1, d), batch, seq)


# speedup vs baseline: 1.1121x; 1.0374x over previous
import functools

import jax
import jax.numpy as jnp
from jax import lax
from jax.experimental import pallas as pl
from jax.experimental.pallas import tpu as pltpu
from jax.experimental.pallas import tpu_sc as plsc

EPS = 1e-12

_NUM_CORES = 2
_NUM_SUBCORES = 16
_NUM_WORKERS = _NUM_CORES * _NUM_SUBCORES
_CHUNK = 128


_NBUF = 5


def _sc_gather(table, idx2d):
    (n,) = idx2d.shape
    chunk = _CHUNK
    d = table.shape[1]
    dt = table.dtype
    rows_pw = n // _NUM_WORKERS
    chunks_pw = rows_pw // chunk
    n_rounds = chunks_pw // _NBUF
    mesh = plsc.VectorSubcoreMesh(core_axis_name="c", subcore_axis_name="s")

    @functools.partial(
        pl.kernel,
        out_type=jax.ShapeDtypeStruct((n, d), dt),
        mesh=mesh,
        scratch_types=[
            pltpu.VMEM((rows_pw,), jnp.int32),
            [pltpu.VMEM((chunk, d), dt) for _ in range(_NBUF)],
            [pltpu.SemaphoreType.DMA for _ in range(_NBUF)],
            [pltpu.SemaphoreType.DMA for _ in range(_NBUF)],
        ],
    )
    def gather_kernel(table_hbm, idx_hbm, out_hbm, idx_v, rows, gsem, wsem):
        wid = lax.axis_index("s") * _NUM_CORES + lax.axis_index("c")
        rbase = wid * rows_pw

        pltpu.sync_copy(idx_hbm.at[pl.ds(rbase, rows_pw)], idx_v)

        def g_start(c, b):
            pltpu.async_copy(table_hbm.at[idx_v.at[pl.ds(c * chunk, chunk)]],
                             rows[b], gsem[b])

        def g_wait(b):
            pltpu.make_async_copy(table_hbm.at[idx_v.at[pl.ds(0, chunk)]],
                                  rows[b], gsem[b]).wait()

        def w_start(c, b):
            pltpu.async_copy(
                rows[b], out_hbm.at[pl.ds(rbase + c * chunk, chunk)], wsem[b])

        def w_wait(b):
            pltpu.make_async_copy(rows[b], out_hbm.at[pl.ds(rbase, chunk)],
                                  wsem[b]).wait()

        for b in range(_NBUF - 1):
            g_start(b, b)
        w_start(_NBUF - 1, _NBUF - 1)

        def round_body(r, carry):
            c0 = r * _NBUF
            for b in range(_NBUF):
                c = c0 + b
                pb = (b + _NBUF - 1) % _NBUF
                w_wait(pb)
                g_start(c + _NBUF - 1, pb)
                g_wait(b)
                w_start(c, b)
            return carry

        lax.fori_loop(0, n_rounds - 1, round_body, 0)

        c0 = (n_rounds - 1) * _NBUF
        w_wait(_NBUF - 1)
        g_start(c0 + _NBUF - 1, _NBUF - 1)
        for b in range(_NBUF):
            g_wait(b)
            w_start(c0 + b, b)
        for b in range(_NBUF):
            w_wait(b)

    return gather_kernel(table, idx2d)


def _tc_dense(feats, gathered, w1, b1, w2, b2, ln_w, ln_b, batch, seq):
    f = feats.shape[-1]
    d = w1.shape[1]
    block_b = 256
    block = block_b * seq
    grid = (batch // block_b,)

    def body(f_ref, g_ref, w1_ref, b1_ref, w2_ref, b2_ref, lw_ref, lb_ref,
             o_ref):
        h = jnp.tanh(
            jnp.dot(f_ref[...].reshape(block, f), w1_ref[...],
                    preferred_element_type=jnp.float32) + b1_ref[...])
        nh = jnp.dot(h, w2_ref[...],
                     preferred_element_type=jnp.float32) + b2_ref[...]
        x = g_ref[...] + nh
        u = jnp.mean(x, axis=-1, keepdims=True)
        xc = x - u
        s = jnp.mean(xc * xc, axis=-1, keepdims=True)
        y = lw_ref[...] * (xc * lax.rsqrt(s + EPS)) + lb_ref[...]
        o_ref[...] = y.reshape(block_b, seq, d)

    fixed = lambda i: (0, 0)
    return pl.pallas_call(
        body,
        grid=grid,
        in_specs=[
            pl.BlockSpec((block_b, seq, f), lambda i: (i, 0, 0)),
            pl.BlockSpec((block, d), lambda i: (i, 0)),
            pl.BlockSpec((f, d), fixed),
            pl.BlockSpec((1, d), fixed),
            pl.BlockSpec((d, d), fixed),
            pl.BlockSpec((1, d), fixed),
            pl.BlockSpec((1, d), fixed),
            pl.BlockSpec((1, d), fixed),
        ],
        out_specs=pl.BlockSpec((block_b, seq, d), lambda i: (i, 0, 0)),
        out_shape=jax.ShapeDtypeStruct((batch, seq, d), jnp.float32),
    )(feats, gathered, w1, b1, w2, b2, ln_w, ln_b)


def kernel(node_ids, node_features, emb_table, W1, b1, W2, b2, ln_weight,
           ln_bias):
    batch, seq = node_ids.shape
    n = batch * seq
    d = emb_table.shape[1]
    idx2d = node_ids.reshape(n).astype(jnp.int32)
    gathered = _sc_gather(emb_table, idx2d)
    return _tc_dense(node_features, gathered, W1, b1.reshape(1, d), W2,
                     b2.reshape(1, d), ln_weight.reshape(1, d),
                     ln_bias.reshape(1, d), batch, seq)
